# baseline (speedup 1.0000x reference)
import functools

import jax
import jax.numpy as jnp
import numpy as np
from jax import lax
from jax.experimental import pallas as pl
from jax.experimental.pallas import tpu as pltpu
from jax.experimental.pallas import tpu_sc as plsc

T_SIZE = 2097152
TBL_MASK = np.int32(T_SIZE - 1)
CELL = np.float32(0.01)
INV_CELL = np.float32(100.0)
T_STEP = np.float32(0.01)
S = 128
N = 4096
NW = 32
RPW = N // NW
G = 16
NG = RPW // G
K = 16
NCH = S // K
DP = 8
HA = np.int32(73856093)
HB = np.int32(19349663)
HC = np.int32(83492791)
RSQRT_MAGIC = np.int32(0x5F3759DF)
F1 = np.float32(1.0)
FH = np.float32(0.5)


def _fld(rayv, iota, gb, i):
    return plsc.load_gather(rayv, [gb + iota, jnp.full((16,), i, jnp.int32)])


def _rsqrt(x):
    y = plsc.bitcast(RSQRT_MAGIC - lax.shift_right_logical(plsc.bitcast(x, jnp.int32), 1),
                     jnp.float32)
    for _ in range(3):
        y = y * (np.float32(1.5) - FH * x * y * y)
    return y


def _sc_body(rays_h, beta_h, table_h,
             rgb_h, dep_h, nrm_h, acc_h, sg_h,
             rayv, betav, idxb0, idxb1, fb0, fb1, rowsb0, rowsb1,
             gradsb, rgbb, depb, nrmb, accb, sem0, sem1):
    cidx = lax.axis_index("c")
    sidx = lax.axis_index("s")
    wid = sidx * 2 + cidx
    r0 = wid * RPW
    iota = lax.iota(jnp.int32, 16)
    zeros = jnp.zeros((16,), jnp.float32)

    pltpu.sync_copy(rays_h.at[pl.ds(r0, RPW)], rayv)
    pltpu.sync_copy(beta_h, betav)

    def group_body(g, _):
        gb = g * G
        o_x = _fld(rayv, iota, gb, 0)
        o_y = _fld(rayv, iota, gb, 1)
        o_z = _fld(rayv, iota, gb, 2)
        d_x = _fld(rayv, iota, gb, 3)
        d_y = _fld(rayv, iota, gb, 4)
        d_z = _fld(rayv, iota, gb, 5)
        near_v = _fld(rayv, iota, gb, 6)
        far_v = _fld(rayv, iota, gb, 7)
        dn_v = _fld(rayv, iota, gb, 8)
        beta_c = jnp.maximum(betav[...], np.float32(1e-4))
        alpha = F1 / beta_c

        def phase_a(ch, idxb, fb):
            s0 = ch * K

            def pa(sl, _):
                sf = (s0 + sl).astype(jnp.float32)
                tn = near_v + sf * T_STEP
                tf_ = tn + T_STEP
                tm = FH * (tn + tf_)
                i0 = []
                for axis, (o_, d_) in enumerate(((o_x, d_x), (o_y, d_y), (o_z, d_z))):
                    p = o_ + tm * d_
                    u = p / CELL
                    ti = u.astype(jnp.int32)
                    tif = ti.astype(jnp.float32)
                    lt = u < tif
                    i0.append(ti - lt.astype(jnp.int32))
                    fb[sl, axis, :] = u - (tif - lt.astype(jnp.float32))
                hx0 = i0[0] * HA
                hx1 = hx0 + HA
                hy0 = i0[1] * HB
                hy1 = hy0 + HB
                hz0 = i0[2] * HC
                hz1 = hz0 + HC
                for dx in (0, 1):
                    hx = hx1 if dx else hx0
                    for dy in (0, 1):
                        hxy = hx ^ (hy1 if dy else hy0)
                        for dz in (0, 1):
                            c = dx * 4 + dy * 2 + dz
                            h = (hxy ^ (hz1 if dz else hz0)) & TBL_MASK
                            idxb[pl.ds((sl * 8 + c) * 16, 16)] = h
                return 0

            lax.fori_loop(0, K, pa, 0)

        def fire(idxb, rowsb, sem):
            pltpu.async_copy(table_h.at[idxb], rowsb, sem)

        def waitd(idxb, rowsb, sem):
            pltpu.make_async_copy(table_h.at[idxb], rowsb, sem).wait()

        def phase_b(ch, rowsb, fb, carry):
            s0 = ch * K

            def pb(sl, carry):
                (cs, cr, cg, cb_, cd, cnx, cny, cnz, ca) = carry
                sf = (s0 + sl).astype(jnp.float32)
                tn = near_v + sf * T_STEP
                tf_ = tn + T_STEP
                tm = FH * (tn + tf_)
                f0 = fb[sl, 0, :]
                f1 = fb[sl, 1, :]
                f2 = fb[sl, 2, :]
                m0 = F1 - f0
                m1 = F1 - f1
                m2 = F1 - f2
                wx = (m0, f0)
                wy = (m1, f1)
                wz = (m2, f2)
                rowbase = sl * 128
                e0 = e1 = e2 = e3 = e4 = zeros
                gx = gy = gz = zeros
                for dx in (0, 1):
                    for dy in (0, 1):
                        wxy = wx[dx] * wy[dy]
                        for dz in (0, 1):
                            c = dx * 4 + dy * 2 + dz
                            rows = rowbase + c * 16 + iota
                            r0c = plsc.load_gather(rowsb, [rows, jnp.zeros((16,), jnp.int32)])
                            wyz = wy[dy] * wz[dz]
                            wxz = wx[dx] * wz[dz]
                            w = wxy * wz[dz]
                            e0 = e0 + w * r0c
                            e1 = e1 + w * plsc.load_gather(rowsb, [rows, jnp.full((16,), 1, jnp.int32)])
                            e2 = e2 + w * plsc.load_gather(rowsb, [rows, jnp.full((16,), 2, jnp.int32)])
                            e3 = e3 + w * plsc.load_gather(rowsb, [rows, jnp.full((16,), 3, jnp.int32)])
                            e4 = e4 + w * plsc.load_gather(rowsb, [rows, jnp.full((16,), 4, jnp.int32)])
                            gx = gx + wyz * r0c if dx else gx - wyz * r0c
                            gy = gy + wxz * r0c if dy else gy - wxz * r0c
                            gz = gz + wxy * r0c if dz else gz - wxy * r0c
                maskf = jnp.where((tm <= far_v) & (e4 >= F1), F1, np.float32(0.0))
                sgx = (gx * INV_CELL) * maskf
                sgy = (gy * INV_CELL) * maskf
                sgz = (gz * INV_CELL) * maskf
                gflat = (iota * S + (s0 + sl)) * 3
                plsc.store_scatter(gradsb, [gflat], sgx)
                plsc.store_scatter(gradsb, [gflat + 1], sgy)
                plsc.store_scatter(gradsb, [gflat + 2], sgz)
                e = jnp.exp(-jnp.abs(e0) / beta_c) - F1
                sigma = FH * alpha * (F1 + jnp.sign(e0) * e) * maskf
                delta = (tf_ - tn) * sigma
                trans = jnp.exp(-cs)
                aa = F1 - jnp.exp(-delta)
                w_ = trans * aa
                n2 = jnp.maximum(sgx * sgx + sgy * sgy + sgz * sgz, np.float32(1e-24))
                rinv = _rsqrt(n2)
                return (cs + delta,
                        cr + w_ * e1, cg + w_ * e2, cb_ + w_ * e3,
                        cd + w_ * tm,
                        cnx + w_ * (sgx * rinv),
                        cny + w_ * (sgy * rinv),
                        cnz + w_ * (sgz * rinv),
                        ca + w_)

            return lax.fori_loop(0, K, pb, carry)

        carry0 = (zeros,) * 9
        phase_a(0, idxb0, fb0)
        fire(idxb0, rowsb0, sem0)

        def body2(i, carry):
            ch0 = 2 * i
            ch1 = ch0 + 1
            phase_a(ch1, idxb1, fb1)
            fire(idxb1, rowsb1, sem1)
            waitd(idxb0, rowsb0, sem0)
            carry = phase_b(ch0, rowsb0, fb0, carry)

            @pl.when(i < NCH // 2 - 1)
            def _():
                phase_a(ch0 + 2, idxb0, fb0)
                fire(idxb0, rowsb0, sem0)

            waitd(idxb1, rowsb1, sem1)
            return phase_b(ch1, rowsb1, fb1, carry)

        (cs, cr, cg, cb_, cd, cnx, cny, cnz, ca) = lax.fori_loop(
            0, NCH // 2, body2, carry0)

        zi = jnp.zeros((16,), jnp.int32)
        plsc.store_scatter(rgbb, [iota, zi], cr)
        plsc.store_scatter(rgbb, [iota, jnp.full((16,), 1, jnp.int32)], cg)
        plsc.store_scatter(rgbb, [iota, jnp.full((16,), 2, jnp.int32)], cb_)
        plsc.store_scatter(nrmb, [iota, zi], cnx)
        plsc.store_scatter(nrmb, [iota, jnp.full((16,), 1, jnp.int32)], cny)
        plsc.store_scatter(nrmb, [iota, jnp.full((16,), 2, jnp.int32)], cnz)
        plsc.store_scatter(depb, [iota, zi], cd / dn_v)
        plsc.store_scatter(accb, [iota, zi], ca)

        row0 = r0 + gb
        pltpu.sync_copy(rgbb, rgb_h.at[pl.ds(row0, G)])
        pltpu.sync_copy(nrmb, nrm_h.at[pl.ds(row0, G)])
        pltpu.sync_copy(depb, dep_h.at[pl.ds(row0, G)])
        pltpu.sync_copy(accb, acc_h.at[pl.ds(row0, G)])
        pltpu.sync_copy(gradsb, sg_h.at[pl.ds(row0 * S * 3, G * S * 3)])
        return 0

    lax.fori_loop(0, NG, group_body, 0)


@functools.partial(
    pl.kernel,
    out_type=(
        jax.ShapeDtypeStruct((N, 3), jnp.float32),
        jax.ShapeDtypeStruct((N, 1), jnp.float32),
        jax.ShapeDtypeStruct((N, 3), jnp.float32),
        jax.ShapeDtypeStruct((N, 1), jnp.float32),
        jax.ShapeDtypeStruct((N * S * 3,), jnp.float32),
    ),
    mesh=plsc.VectorSubcoreMesh(core_axis_name="c", subcore_axis_name="s"),
    compiler_params=pltpu.CompilerParams(
        needs_layout_passes=False, use_tc_tiling_on_sc=False),
    scratch_types=[
        pltpu.VMEM((RPW, 9), jnp.float32),
        pltpu.VMEM((16,), jnp.float32),
        pltpu.VMEM((K * 128,), jnp.int32),
        pltpu.VMEM((K * 128,), jnp.int32),
        pltpu.VMEM((K, 3, 16), jnp.float32),
        pltpu.VMEM((K, 3, 16), jnp.float32),
        pltpu.VMEM((K * 128, DP), jnp.float32),
        pltpu.VMEM((K * 128, DP), jnp.float32),
        pltpu.VMEM((G * S * 3,), jnp.float32),
        pltpu.VMEM((G, 3), jnp.float32),
        pltpu.VMEM((G, 1), jnp.float32),
        pltpu.VMEM((G, 3), jnp.float32),
        pltpu.VMEM((G, 1), jnp.float32),
        pltpu.SemaphoreType.DMA,
        pltpu.SemaphoreType.DMA,
    ],
)
def _voxels_sc(rays_h, beta_h, table_h, rgb_h, dep_h, nrm_h, acc_h, sg_h,
               rayv, betav, idxb0, idxb1, fb0, fb1, rowsb0, rowsb1,
               gradsb, rgbb, depb, nrmb, accb, sem0, sem1):
    _sc_body(rays_h, beta_h, table_h,
             rgb_h, dep_h, nrm_h, acc_h, sg_h,
             rayv, betav, idxb0, idxb1, fb0, fb1, rowsb0, rowsb1,
             gradsb, rgbb, depb, nrmb, accb, sem0, sem1)


def kernel(rays_o, rays_d, rays_d_norm, near, far, table, beta):
    rays_pack = jnp.concatenate(
        [rays_o, rays_d, near, far, rays_d_norm], axis=1).astype(jnp.float32)
    beta16 = jnp.broadcast_to(beta.reshape(-1)[:1], (16,)).astype(jnp.float32)
    table8 = jnp.concatenate(
        [table, jnp.zeros((T_SIZE, DP - 5), jnp.float32)], axis=1)
    rgb, dep, nrm, acc, sgf = _voxels_sc(rays_pack, beta16, table8)
    return rgb, dep, nrm, acc, sgf.reshape(N * S, 3)


# speedup vs baseline: 1.3443x; 1.3443x over previous
import functools

import jax
import jax.numpy as jnp
import numpy as np
from jax import lax
from jax.experimental import pallas as pl
from jax.experimental.pallas import tpu as pltpu
from jax.experimental.pallas import tpu_sc as plsc

T_SIZE = 2097152
TBL_MASK = np.int32(T_SIZE - 1)
CELL = np.float32(0.01)
INV_CELL = np.float32(100.0)
T_STEP = np.float32(0.01)
S = 128
N = 4096
NW = 32
RPW = N // NW
G = 16
NG = RPW // G
K = 16
NCH = S // K
DP = 8
NROWS = T_SIZE * 5 // DP
MAXQ = np.int32(NROWS - 1)
HA = np.int32(73856093)
HB = np.int32(19349663)
HC = np.int32(83492791)
RSQRT_MAGIC = np.int32(0x5F3759DF)
F1 = np.float32(1.0)
FH = np.float32(0.5)


def _fld(rayv, iota, gb, i):
    return plsc.load_gather(rayv, [gb + iota, jnp.full((16,), i, jnp.int32)])


def _rsqrt(x):
    y = plsc.bitcast(RSQRT_MAGIC - lax.shift_right_logical(plsc.bitcast(x, jnp.int32), 1),
                     jnp.float32)
    for _ in range(3):
        y = y * (np.float32(1.5) - FH * x * y * y)
    return y


def _sc_body(rays_h, beta_h, table_h,
             rgb_h, dep_h, nrm_h, acc_h, sg_h,
             rayv, betav, idxb0, idxb1, fb0, fb1, offb0, offb1, rowsb0, rowsb1,
             gradsb, rgbb, depb, nrmb, accb, sem0, sem1):
    cidx = lax.axis_index("c")
    sidx = lax.axis_index("s")
    wid = sidx * 2 + cidx
    r0 = wid * RPW
    iota = lax.iota(jnp.int32, 16)
    zeros = jnp.zeros((16,), jnp.float32)

    pltpu.sync_copy(rays_h.at[pl.ds(r0, RPW)], rayv)
    pltpu.sync_copy(beta_h, betav)

    def group_body(g, _):
        gb = g * G
        o_x = _fld(rayv, iota, gb, 0)
        o_y = _fld(rayv, iota, gb, 1)
        o_z = _fld(rayv, iota, gb, 2)
        d_x = _fld(rayv, iota, gb, 3)
        d_y = _fld(rayv, iota, gb, 4)
        d_z = _fld(rayv, iota, gb, 5)
        near_v = _fld(rayv, iota, gb, 6)
        far_v = _fld(rayv, iota, gb, 7)
        dn_v = _fld(rayv, iota, gb, 8)
        beta_c = jnp.maximum(betav[...], np.float32(1e-4))
        alpha = F1 / beta_c

        def phase_a(ch, idxb, fb, offb):
            s0 = ch * K

            def pa(sl, _):
                sf = (s0 + sl).astype(jnp.float32)
                tn = near_v + sf * T_STEP
                tf_ = tn + T_STEP
                tm = FH * (tn + tf_)
                i0 = []
                for axis, (o_, d_) in enumerate(((o_x, d_x), (o_y, d_y), (o_z, d_z))):
                    p = o_ + tm * d_
                    u = p / CELL
                    ti = u.astype(jnp.int32)
                    tif = ti.astype(jnp.float32)
                    lt = u < tif
                    i0.append(ti - lt.astype(jnp.int32))
                    fb[sl, axis, :] = u - (tif - lt.astype(jnp.float32))
                hx0 = i0[0] * HA
                hx1 = hx0 + HA
                hy0 = i0[1] * HB
                hy1 = hy0 + HB
                hz0 = i0[2] * HC
                hz1 = hz0 + HC
                for dx in (0, 1):
                    hx = hx1 if dx else hx0
                    for dy in (0, 1):
                        hxy = hx ^ (hy1 if dy else hy0)
                        for dz in (0, 1):
                            c = dx * 4 + dy * 2 + dz
                            h = (hxy ^ (hz1 if dz else hz0)) & TBL_MASK
                            w0 = h * 5
                            q = lax.shift_right_logical(w0, 1 + 2)
                            off = w0 & 7
                            q1 = jnp.minimum(q + 1, MAXQ)
                            ebase = (sl * 8 + c) * 32 + iota * 2
                            plsc.store_scatter(idxb, [ebase], q)
                            plsc.store_scatter(idxb, [ebase + 1], q1)
                            offb[sl, c, :] = off
                return 0

            lax.fori_loop(0, K, pa, 0)

        def fire(idxb, rowsb, sem):
            pltpu.async_copy(table_h.at[idxb], rowsb, sem)

        def waitd(idxb, rowsb, sem):
            pltpu.make_async_copy(table_h.at[idxb], rowsb, sem).wait()

        def phase_b(ch, rowsb, fb, offb, carry):
            s0 = ch * K

            def pb(sl, carry):
                (cs, cr, cg, cb_, cd, cnx, cny, cnz, ca) = carry
                sf = (s0 + sl).astype(jnp.float32)
                tn = near_v + sf * T_STEP
                tf_ = tn + T_STEP
                tm = FH * (tn + tf_)
                f0 = fb[sl, 0, :]
                f1 = fb[sl, 1, :]
                f2 = fb[sl, 2, :]
                m0 = F1 - f0
                m1 = F1 - f1
                m2 = F1 - f2
                wx = (m0, f0)
                wy = (m1, f1)
                wz = (m2, f2)
                rowbase = sl * 128
                e0 = e1 = e2 = e3 = e4 = zeros
                gx = gy = gz = zeros
                seven = jnp.full((16,), 7, jnp.int32)
                for dx in (0, 1):
                    for dy in (0, 1):
                        wxy = wx[dx] * wy[dy]
                        for dz in (0, 1):
                            c = dx * 4 + dy * 2 + dz
                            basew = (rowbase + c * 16 + iota) * 16 + offb[sl, c, :]

                            def ld(j, basew=basew):
                                w_ = basew + j
                                return plsc.load_gather(
                                    rowsb,
                                    [lax.shift_right_logical(w_, 3), w_ & seven])

                            r0c = ld(0)
                            wyz = wy[dy] * wz[dz]
                            wxz = wx[dx] * wz[dz]
                            w = wxy * wz[dz]
                            e0 = e0 + w * r0c
                            e1 = e1 + w * ld(1)
                            e2 = e2 + w * ld(2)
                            e3 = e3 + w * ld(3)
                            e4 = e4 + w * ld(4)
                            gx = gx + wyz * r0c if dx else gx - wyz * r0c
                            gy = gy + wxz * r0c if dy else gy - wxz * r0c
                            gz = gz + wxy * r0c if dz else gz - wxy * r0c
                maskf = jnp.where((tm <= far_v) & (e4 >= F1), F1, np.float32(0.0))
                sgx = (gx * INV_CELL) * maskf
                sgy = (gy * INV_CELL) * maskf
                sgz = (gz * INV_CELL) * maskf
                grows = iota * S + (s0 + sl)
                plsc.store_scatter(gradsb, [grows, jnp.zeros((16,), jnp.int32)], sgx)
                plsc.store_scatter(gradsb, [grows, jnp.full((16,), 1, jnp.int32)], sgy)
                plsc.store_scatter(gradsb, [grows, jnp.full((16,), 2, jnp.int32)], sgz)
                e = jnp.exp(-jnp.abs(e0) / beta_c) - F1
                sigma = FH * alpha * (F1 + jnp.sign(e0) * e) * maskf
                delta = (tf_ - tn) * sigma
                trans = jnp.exp(-cs)
                aa = F1 - jnp.exp(-delta)
                w_ = trans * aa
                n2 = jnp.maximum(sgx * sgx + sgy * sgy + sgz * sgz, np.float32(1e-24))
                rinv = _rsqrt(n2)
                return (cs + delta,
                        cr + w_ * e1, cg + w_ * e2, cb_ + w_ * e3,
                        cd + w_ * tm,
                        cnx + w_ * (sgx * rinv),
                        cny + w_ * (sgy * rinv),
                        cnz + w_ * (sgz * rinv),
                        ca + w_)

            return lax.fori_loop(0, K, pb, carry)

        carry0 = (zeros,) * 9
        phase_a(0, idxb0, fb0, offb0)
        fire(idxb0, rowsb0, sem0)

        def body2(i, carry):
            ch0 = 2 * i
            ch1 = ch0 + 1
            phase_a(ch1, idxb1, fb1, offb1)
            fire(idxb1, rowsb1, sem1)
            waitd(idxb0, rowsb0, sem0)
            carry = phase_b(ch0, rowsb0, fb0, offb0, carry)

            @pl.when(i < NCH // 2 - 1)
            def _():
                phase_a(ch0 + 2, idxb0, fb0, offb0)
                fire(idxb0, rowsb0, sem0)

            waitd(idxb1, rowsb1, sem1)
            return phase_b(ch1, rowsb1, fb1, offb1, carry)

        (cs, cr, cg, cb_, cd, cnx, cny, cnz, ca) = lax.fori_loop(
            0, NCH // 2, body2, carry0)

        zi = jnp.zeros((16,), jnp.int32)
        plsc.store_scatter(rgbb, [iota, zi], cr)
        plsc.store_scatter(rgbb, [iota, jnp.full((16,), 1, jnp.int32)], cg)
        plsc.store_scatter(rgbb, [iota, jnp.full((16,), 2, jnp.int32)], cb_)
        plsc.store_scatter(nrmb, [iota, zi], cnx)
        plsc.store_scatter(nrmb, [iota, jnp.full((16,), 1, jnp.int32)], cny)
        plsc.store_scatter(nrmb, [iota, jnp.full((16,), 2, jnp.int32)], cnz)
        plsc.store_scatter(depb, [iota, zi], cd / dn_v)
        plsc.store_scatter(accb, [iota, zi], ca)

        row0 = r0 + gb
        pltpu.sync_copy(rgbb, rgb_h.at[pl.ds(row0, G)])
        pltpu.sync_copy(nrmb, nrm_h.at[pl.ds(row0, G)])
        pltpu.sync_copy(depb, dep_h.at[pl.ds(row0, G)])
        pltpu.sync_copy(accb, acc_h.at[pl.ds(row0, G)])
        pltpu.sync_copy(gradsb, sg_h.at[pl.ds(row0 * S, G * S)])
        return 0

    lax.fori_loop(0, NG, group_body, 0)


@functools.partial(
    pl.kernel,
    out_type=(
        jax.ShapeDtypeStruct((N, 3), jnp.float32),
        jax.ShapeDtypeStruct((N, 1), jnp.float32),
        jax.ShapeDtypeStruct((N, 3), jnp.float32),
        jax.ShapeDtypeStruct((N, 1), jnp.float32),
        jax.ShapeDtypeStruct((N * S, 3), jnp.float32),
    ),
    mesh=plsc.VectorSubcoreMesh(core_axis_name="c", subcore_axis_name="s"),
    compiler_params=pltpu.CompilerParams(
        needs_layout_passes=False, use_tc_tiling_on_sc=False),
    scratch_types=[
        pltpu.VMEM((RPW, 9), jnp.float32),
        pltpu.VMEM((16,), jnp.float32),
        pltpu.VMEM((K * 256,), jnp.int32),
        pltpu.VMEM((K * 256,), jnp.int32),
        pltpu.VMEM((K, 3, 16), jnp.float32),
        pltpu.VMEM((K, 3, 16), jnp.float32),
        pltpu.VMEM((K, 8, 16), jnp.int32),
        pltpu.VMEM((K, 8, 16), jnp.int32),
        pltpu.VMEM((K * 256, DP), jnp.float32),
        pltpu.VMEM((K * 256, DP), jnp.float32),
        pltpu.VMEM((G * S, 3), jnp.float32),
        pltpu.VMEM((G, 3), jnp.float32),
        pltpu.VMEM((G, 1), jnp.float32),
        pltpu.VMEM((G, 3), jnp.float32),
        pltpu.VMEM((G, 1), jnp.float32),
        pltpu.SemaphoreType.DMA,
        pltpu.SemaphoreType.DMA,
    ],
)
def _voxels_sc(rays_h, beta_h, table_h, rgb_h, dep_h, nrm_h, acc_h, sg_h,
               rayv, betav, idxb0, idxb1, fb0, fb1, offb0, offb1, rowsb0, rowsb1,
               gradsb, rgbb, depb, nrmb, accb, sem0, sem1):
    _sc_body(rays_h, beta_h, table_h,
             rgb_h, dep_h, nrm_h, acc_h, sg_h,
             rayv, betav, idxb0, idxb1, fb0, fb1, offb0, offb1, rowsb0, rowsb1,
             gradsb, rgbb, depb, nrmb, accb, sem0, sem1)


def kernel(rays_o, rays_d, rays_d_norm, near, far, table, beta):
    rays_pack = jnp.concatenate(
        [rays_o, rays_d, near, far, rays_d_norm], axis=1).astype(jnp.float32)
    beta16 = jnp.broadcast_to(beta.reshape(-1)[:1], (16,)).astype(jnp.float32)
    tablev = table.reshape(NROWS, DP)
    return _voxels_sc(rays_pack, beta16, tablev)
